# baseline (speedup 1.0000x reference)
import functools

import jax
import jax.numpy as jnp
from jax.experimental import pallas as pl
from jax.experimental.pallas import tpu as pltpu


def _logits_kernel(x_ref, w1_ref, b1_ref, w2t_ref, b2_ref, out_ref, *, K):
    h = jnp.tanh(
        jnp.dot(x_ref[...], w1_ref[...], preferred_element_type=jnp.float32)
        + b1_ref[...]
    )
    w2t = w2t_ref[...]
    cols = []
    for k in range(K):
        col = jnp.sum(h * w2t[k:k + 1, :], axis=1, keepdims=True)
        cols.append(col + b2_ref[0, k])
    a = cols[0] if K == 1 else jnp.concatenate(cols, axis=1)
    out_ref[...] = a.astype(out_ref.dtype)


def _softmax_all_kernel(a_ref, out_ref):
    a = a_ref[...]
    m = jnp.max(a)
    e = jnp.exp(a - m)
    out_ref[...] = e / jnp.sum(e)


def _softmax_axis0_kernel(a_ref, out_ref):
    a = a_ref[...]
    m = jnp.max(a, axis=0, keepdims=True)
    e = jnp.exp(a - m)
    out_ref[...] = e / jnp.sum(e, axis=0, keepdims=True)


def kernel(x, w1, b1, w2, b2):
    N, L = x.shape
    D = w1.shape[1]
    K = w2.shape[1]

    x = jnp.asarray(x, jnp.float32)
    w1 = jnp.asarray(w1, jnp.float32)
    b1 = jnp.asarray(b1, jnp.float32).reshape(1, D)
    w2t = jnp.asarray(w2, jnp.float32).T.reshape(K, D)
    b2s = jnp.asarray(b2, jnp.float32).reshape(1, K)

    block_n = next((t for t in (4096, 2048, 1024, 512, 256, 128, 64, 32, 16, 8)
                    if N % t == 0), N)
    num_tiles = N // block_n

    cost = pl.CostEstimate(
        flops=2 * N * L * D + 2 * N * D * K,
        transcendentals=N * D,
        bytes_accessed=4 * (N * L + L * D + D + D * K + K + N * K),
    )

    logits = pl.pallas_call(
        functools.partial(_logits_kernel, K=K),
        out_shape=jax.ShapeDtypeStruct((N, K), jnp.float32),
        grid=(num_tiles,),
        in_specs=[
            pl.BlockSpec((block_n, L), lambda i: (i, 0)),
            pl.BlockSpec((L, D), lambda i: (0, 0)),
            pl.BlockSpec((1, D), lambda i: (0, 0)),
            pl.BlockSpec((K, D), lambda i: (0, 0)),
            pl.BlockSpec(memory_space=pltpu.MemorySpace.SMEM),
        ],
        out_specs=pl.BlockSpec((block_n, K), lambda i: (i, 0)),
        compiler_params=pltpu.CompilerParams(
            dimension_semantics=("arbitrary",),
        ),
        cost_estimate=cost,
    )(x, w1, b1, w2t, b2s)

    if K == 1 and N % 128 == 0:
        rows = N // 128
        out = pl.pallas_call(
            _softmax_all_kernel,
            out_shape=jax.ShapeDtypeStruct((rows, 128), jnp.float32),
        )(logits.reshape(rows, 128))
        return out.reshape(K, N)
    out = pl.pallas_call(
        _softmax_axis0_kernel,
        out_shape=jax.ShapeDtypeStruct((N, K), jnp.float32),
    )(logits)
    return out.T


# speedup vs baseline: 1.5640x; 1.5640x over previous
import functools

import jax
import jax.numpy as jnp
from jax.experimental import pallas as pl
from jax.experimental.pallas import tpu as pltpu


def _fused_kernel(x_ref, w1_ref, b1_ref, w2t_ref, b2_ref, out_ref, *,
                  block_n, K):
    i = pl.program_id(0)
    h = jnp.tanh(
        jnp.dot(x_ref[...], w1_ref[...], preferred_element_type=jnp.float32)
        + b1_ref[...]
    )
    at = jax.lax.dot_general(
        w2t_ref[...], h, (((1,), (1,)), ((), ())),
        preferred_element_type=jnp.float32,
    ) + b2_ref[...]
    out_ref[:, pl.ds(i * block_n, block_n)] = at

    @pl.when(i == pl.num_programs(0) - 1)
    def _finalize():
        a = out_ref[...]
        m = jnp.max(a, axis=1, keepdims=True)
        e = jnp.exp(a - m)
        out_ref[...] = e / jnp.sum(e, axis=1, keepdims=True)


def kernel(x, w1, b1, w2, b2):
    N, L = x.shape
    D = w1.shape[1]
    K = w2.shape[1]

    x = jnp.asarray(x, jnp.float32)
    w1 = jnp.asarray(w1, jnp.float32)
    b1 = jnp.asarray(b1, jnp.float32).reshape(1, D)
    w2t = jnp.asarray(w2, jnp.float32).T.reshape(K, D)
    b2c = jnp.asarray(b2, jnp.float32).reshape(K, 1)

    block_n = next((t for t in (4096, 2048, 1024, 512, 256, 128, 64, 32, 16, 8)
                    if N % t == 0), N)
    num_tiles = N // block_n

    cost = pl.CostEstimate(
        flops=2 * N * L * D + 2 * N * D * K + 6 * N * K,
        transcendentals=N * D + N * K,
        bytes_accessed=4 * (N * L + L * D + D + D * K + K + N * K),
    )

    out = pl.pallas_call(
        functools.partial(_fused_kernel, block_n=block_n, K=K),
        out_shape=jax.ShapeDtypeStruct((K, N), jnp.float32),
        grid=(num_tiles,),
        in_specs=[
            pl.BlockSpec((block_n, L), lambda i: (i, 0)),
            pl.BlockSpec((L, D), lambda i: (0, 0)),
            pl.BlockSpec((1, D), lambda i: (0, 0)),
            pl.BlockSpec((K, D), lambda i: (0, 0)),
            pl.BlockSpec((K, 1), lambda i: (0, 0)),
        ],
        out_specs=pl.BlockSpec((K, N), lambda i: (0, 0)),
        compiler_params=pltpu.CompilerParams(
            dimension_semantics=("arbitrary",),
        ),
        cost_estimate=cost,
    )(x, w1, b1, w2t, b2c)
    return out
